# Initial kernel scaffold; baseline (speedup 1.0000x reference)
import jax
import jax.numpy as jnp
from jax.experimental import pallas as pl


def kernel(x, edge_index, W1, b1, W2, b2, W3, b3, W4, b4):
    raise NotImplementedError("write your pallas kernel here")


# speedup vs baseline: 10.4669x; 10.4669x over previous
import functools

import jax
import jax.numpy as jnp
from jax import lax
from jax.experimental import pallas as pl
from jax.experimental.pallas import tpu as pltpu
from jax.experimental.pallas import tpu_sc as plsc

N_NODES = 100000
N_EDGES = 1600000
NP = 102400
NT = 2 * NP
CHUNK = 128
NCHUNKS = 12800
EP = NCHUNKS * CHUNK
NC = 2
NS = 16
W = 16
ROWS_PER_TILE = NP // NS
ZROWS = 800

MODE_DEG = 0
MODE_EDGE = 1
MODE_PANEL = 2

_mesh = plsc.VectorSubcoreMesh(core_axis_name="c", subcore_axis_name="s")


@functools.partial(
    pl.kernel,
    out_type=jax.ShapeDtypeStruct((NC, NP, W), jnp.float32),
    mesh=_mesh,
    compiler_params=pltpu.CompilerParams(use_tc_tiling_on_sc=False),
    scratch_types=[
        pltpu.VMEM_SHARED((NP, W), jnp.float32),
        pltpu.VMEM((ZROWS, W), jnp.float32),
        pltpu.VMEM((1, CHUNK), jnp.int32),
        pltpu.VMEM((1, CHUNK), jnp.int32),
        pltpu.VMEM((CHUNK, W), jnp.float32),
        pltpu.VMEM((CHUNK, W), jnp.float32),
        pltpu.VMEM((1, W), jnp.int32),
        pltpu.SemaphoreType.DMA,
    ],
)
def _sc_pass(params_hbm, g_hbm, src_hbm, dst_hbm, out_hbm,
             acc, zbuf, isrc, idst, rows, ones, pbuf, sem):
    c = lax.axis_index("c")
    s = lax.axis_index("s")
    pltpu.sync_copy(params_hbm, pbuf)
    mode = pbuf[0][0]
    is_deg = mode == MODE_DEG
    is_panel = mode == MODE_PANEL

    def orow(i, _):
        ones[i] = jnp.ones((W,), jnp.float32)
        return 0
    lax.fori_loop(0, CHUNK, orow, 0)

    def zrow(i, _):
        zbuf[i] = jnp.zeros((W,), jnp.float32)
        return 0
    lax.fori_loop(0, ZROWS, zrow, 0)
    base_row = s * ROWS_PER_TILE
    def zcopy(i, _):
        pltpu.sync_copy(zbuf, acc.at[pl.ds(base_row + i * ZROWS, ZROWS)])
        return 0
    lax.fori_loop(0, ROWS_PER_TILE // ZROWS, zcopy, 0)
    plsc.subcore_barrier()

    cpt_edge = NCHUNKS // (NC * NS)
    cpt_panel = NCHUNKS // NS
    n_chunks = jnp.where(is_panel, cpt_panel, cpt_edge)
    base_chunk = jnp.where(is_panel, s * cpt_panel, (s * NC + c) * cpt_edge)
    off = jnp.where(is_panel, c * NP, 0).astype(jnp.int32)

    def body(j, _):
        ch = base_chunk + j
        pltpu.sync_copy(src_hbm.at[pl.ds(ch, 1)], isrc)
        pltpu.sync_copy(dst_hbm.at[pl.ds(ch, 1)], idst)

        @pl.when(jnp.logical_not(is_deg))
        def _():
            for i in range(CHUNK // 16):
                v = isrc[0, pl.ds(i * 16, 16)]
                isrc[0, pl.ds(i * 16, 16)] = v + off
            pltpu.async_copy(g_hbm.at[isrc.at[0]], rows, sem).wait()
            pltpu.sync_copy(rows, acc.at[idst.at[0]], add=True)

        @pl.when(is_deg)
        def _():
            pltpu.sync_copy(ones, acc.at[idst.at[0]], add=True)

        return 0

    lax.fori_loop(0, n_chunks, body, 0)
    plsc.subcore_barrier()
    pltpu.sync_copy(acc.at[pl.ds(base_row, ROWS_PER_TILE)],
                    out_hbm.at[c].at[pl.ds(base_row, ROWS_PER_TILE)])


BLK = 2048
GRID = NP // BLK


def _row_spec(width):
    return pl.BlockSpec((BLK, width), lambda i: (i, 0))


def _pair_spec(width):
    return pl.BlockSpec((NC, BLK, width), lambda i: (0, i, 0))


_vec_spec = pl.BlockSpec((BLK,), lambda i: (i,))


def _full(a):
    return pl.BlockSpec(a.shape, lambda i: tuple(0 for _ in a.shape))


def _tc0_body(degp_ref, xp_ref, dinv_ref, g1_ref):
    deg = degp_ref[0, :, 0] + degp_ref[1, :, 0] + 1.0
    dinv = lax.rsqrt(deg)
    dinv_ref[...] = dinv
    g1_ref[...] = xp_ref[...] * dinv[:, None]


def _tc1_body(s1_ref, g1_ref, dinv_ref, w1_ref, b1_ref, w2_ref, g2_ref):
    dinv = dinv_ref[...]
    u = (s1_ref[0] + s1_ref[1] + g1_ref[...]) * dinv[:, None]
    h1 = jax.nn.relu(jnp.dot(u, w1_ref[...].T,
                             preferred_element_type=jnp.float32) + b1_ref[...])
    t2 = jnp.dot(h1, w2_ref[...].T, preferred_element_type=jnp.float32)
    g2 = t2 * dinv[:, None]
    g2_ref[0] = g2[:, :W]
    g2_ref[1] = g2[:, W:]


def _tc2_body(s2_ref, g2_ref, dinv_ref, b2_ref, w3_ref, g3_ref):
    dinv = dinv_ref[...]
    ya = (s2_ref[0] + g2_ref[0]) * dinv[:, None] + b2_ref[...][None, :W]
    yb = (s2_ref[1] + g2_ref[1]) * dinv[:, None] + b2_ref[...][None, W:]
    h2 = jax.nn.relu(jnp.concatenate([ya, yb], axis=1))
    t3 = jnp.dot(h2, w3_ref[...].T, preferred_element_type=jnp.float32)
    g3_ref[...] = t3 * dinv[:, None]


def _tc3_body(s3_ref, g3_ref, dinv_ref, b3_ref, w4_ref, g4_ref):
    dinv = dinv_ref[...]
    h3 = jax.nn.relu((s3_ref[0] + s3_ref[1] + g3_ref[...]) * dinv[:, None]
                     + b3_ref[...][None, :])
    t4 = jnp.dot(h3, w4_ref[...].T, preferred_element_type=jnp.float32)
    g4 = t4 * dinv[:, None]
    g4_ref[...] = jnp.concatenate(
        [g4, jnp.zeros((BLK, W - 8), jnp.float32)], axis=1)


def _tc4_body(s4_ref, g4_ref, dinv_ref, b4_ref, out_ref):
    dinv = dinv_ref[...]
    out_ref[...] = ((s4_ref[0, :, :8] + s4_ref[1, :, :8] + g4_ref[:, :8])
                    * dinv[:, None] + b4_ref[...][None, :])


def _params(mode):
    return jnp.full((1, W), mode, jnp.int32)


def kernel(x, edge_index, W1, b1, W2, b2, W3, b3, W4, b4):
    f32 = jnp.float32
    x_pad = jnp.zeros((NT, W), f32).at[:N_NODES, :11].set(x)
    pad = jnp.full((2, EP - N_EDGES), N_NODES, jnp.int32)
    epad = jnp.concatenate([edge_index.astype(jnp.int32), pad], axis=1)
    src2d = epad[0].reshape(NCHUNKS, CHUNK)
    dst2d = epad[1].reshape(NCHUNKS, CHUNK)
    W1p = jnp.pad(W1, ((0, 0), (0, W - 11)))

    degp = _sc_pass(_params(MODE_DEG), x_pad, src2d, dst2d)
    dinv, g1 = pl.pallas_call(
        _tc0_body,
        grid=(GRID,),
        in_specs=[_pair_spec(W), _row_spec(W)],
        out_specs=[_vec_spec, _row_spec(W)],
        out_shape=[jax.ShapeDtypeStruct((NP,), f32),
                   jax.ShapeDtypeStruct((NT, W), f32)],
    )(degp, x_pad)

    s1 = _sc_pass(_params(MODE_EDGE), g1, src2d, dst2d)
    g2 = pl.pallas_call(
        _tc1_body,
        grid=(GRID,),
        in_specs=[_pair_spec(W), _row_spec(W), _vec_spec,
                  _full(W1p), _full(b1), _full(W2)],
        out_specs=_pair_spec(W),
        out_shape=jax.ShapeDtypeStruct((NC, NP, W), f32),
    )(s1, g1, dinv, W1p, b1, W2)

    s2 = _sc_pass(_params(MODE_PANEL), g2.reshape(NT, W), src2d, dst2d)
    g3 = pl.pallas_call(
        _tc2_body,
        grid=(GRID,),
        in_specs=[_pair_spec(W), _pair_spec(W), _vec_spec,
                  _full(b2), _full(W3)],
        out_specs=_row_spec(W),
        out_shape=jax.ShapeDtypeStruct((NT, W), f32),
    )(s2, g2, dinv, b2, W3)

    s3 = _sc_pass(_params(MODE_EDGE), g3, src2d, dst2d)
    g4 = pl.pallas_call(
        _tc3_body,
        grid=(GRID,),
        in_specs=[_pair_spec(W), _row_spec(W), _vec_spec,
                  _full(b3), _full(W4)],
        out_specs=_row_spec(W),
        out_shape=jax.ShapeDtypeStruct((NT, W), f32),
    )(s3, g3, dinv, b3, W4)

    s4 = _sc_pass(_params(MODE_EDGE), g4, src2d, dst2d)
    out = pl.pallas_call(
        _tc4_body,
        grid=(GRID,),
        in_specs=[_pair_spec(W), _row_spec(W), _vec_spec, _full(b4)],
        out_specs=_row_spec(8),
        out_shape=jax.ShapeDtypeStruct((NP, 8), f32),
    )(s4, g4, dinv, b4)

    return out[:N_NODES]


# speedup vs baseline: 22.6185x; 2.1609x over previous
import functools

import jax
import jax.numpy as jnp
from jax import lax
from jax.experimental import pallas as pl
from jax.experimental.pallas import tpu as pltpu
from jax.experimental.pallas import tpu_sc as plsc

N_NODES = 100000
N_EDGES = 1600000
NP = 100352
NT = 2 * NP
CHUNK = 128
NCHUNKS = 12800
EP = NCHUNKS * CHUNK
NC = 2
NS = 16
W = 16
ROWS_PER_TILE = NP // NS
ZROWS = 224

MODE_DEG = 0
MODE_EDGE = 1
MODE_PANEL = 2

_mesh = plsc.VectorSubcoreMesh(core_axis_name="c", subcore_axis_name="s")


NB = 5


@functools.partial(
    pl.kernel,
    out_type=jax.ShapeDtypeStruct((NC, NP, W), jnp.float32),
    mesh=_mesh,
    compiler_params=pltpu.CompilerParams(use_tc_tiling_on_sc=False),
    scratch_types=[
        pltpu.VMEM_SHARED((NP, W), jnp.float32),
        pltpu.VMEM((ZROWS, W), jnp.float32),
        pltpu.VMEM((2, NB, CHUNK), jnp.int32),
        pltpu.VMEM((2, NB, CHUNK), jnp.int32),
        pltpu.VMEM((2, NB, CHUNK, W), jnp.float32),
        pltpu.VMEM((CHUNK, W), jnp.float32),
        pltpu.VMEM((1, W), jnp.int32),
        pltpu.SemaphoreType.DMA,
        pltpu.SemaphoreType.DMA,
        pltpu.SemaphoreType.DMA,
        pltpu.SemaphoreType.DMA,
    ],
)
def _sc_pass(params_hbm, g_hbm, src_hbm, dst_hbm, out_hbm,
             acc, zbuf, isrc, idst, rows, ones, pbuf, semI1, semI2,
             semG, semS):
    c = lax.axis_index("c")
    s = lax.axis_index("s")
    pltpu.sync_copy(params_hbm, pbuf)
    mode = pbuf[0][0]
    is_deg = mode == MODE_DEG
    not_deg = jnp.logical_not(is_deg)
    is_panel = mode == MODE_PANEL

    def orow(i, _):
        ones[i] = jnp.ones((W,), jnp.float32)
        return 0
    lax.fori_loop(0, CHUNK, orow, 0)

    def zrow(i, _):
        zbuf[i] = jnp.zeros((W,), jnp.float32)
        return 0
    lax.fori_loop(0, ZROWS, zrow, 0)
    base_row = s * ROWS_PER_TILE
    def zcopy(i, _):
        pltpu.sync_copy(zbuf, acc.at[pl.ds(base_row + i * ZROWS, ZROWS)])
        return 0
    lax.fori_loop(0, ROWS_PER_TILE // ZROWS, zcopy, 0)

    cpt_edge = NCHUNKS // (NC * NS)
    cpt_panel = NCHUNKS // NS
    base_chunk = jnp.where(is_panel, s * cpt_panel, (s * NC + c) * cpt_edge)
    ng = jnp.where(is_panel, cpt_panel // NB, cpt_edge // NB)
    off = jnp.where(is_panel, c * NP, 0).astype(jnp.int32)

    def issue_isrc(g, p):
        pltpu.async_copy(src_hbm.at[pl.ds(base_chunk + g * NB, NB)],
                         isrc.at[p], semI1)

    def issue_idst(g, p):
        pltpu.async_copy(dst_hbm.at[pl.ds(base_chunk + g * NB, NB)],
                         idst.at[p], semI2)

    def wait_isrc(p):
        pltpu.make_async_copy(src_hbm.at[pl.ds(0, NB)], isrc.at[p],
                              semI1).wait()

        @pl.when(is_panel)
        def _():
            for b in range(NB):
                for i in range(CHUNK // 16):
                    v = isrc[p, b, pl.ds(i * 16, 16)]
                    isrc[p, b, pl.ds(i * 16, 16)] = v + off

    def wait_idst(p):
        pltpu.make_async_copy(dst_hbm.at[pl.ds(0, NB)], idst.at[p],
                              semI2).wait()

    def issue_gathers(p):
        for b in range(NB):
            pltpu.async_copy(g_hbm.at[isrc.at[p].at[b]], rows.at[p].at[b],
                             semG)

    def wait_one_gather(p, b):
        pltpu.make_async_copy(g_hbm.at[pl.ds(0, CHUNK)], rows.at[p].at[b],
                              semG).wait()

    def drain_scatters(p):
        for b in range(NB):
            pltpu.make_async_copy(g_hbm.at[pl.ds(0, CHUNK)], rows.at[p].at[b],
                                  semS).wait()

    @pl.when(not_deg)
    def _():
        issue_isrc(0, 0)
        wait_isrc(0)
        issue_gathers(0)
    issue_idst(0, 0)
    wait_idst(0)
    plsc.subcore_barrier()

    def group(g, p):
        more = g + 1 < ng

        @pl.when(jnp.logical_and(more, not_deg))
        def _():
            issue_isrc(g + 1, 1 - p)

        @pl.when(g > 0)
        def _():
            wait_idst(p)

        for b in range(NB):
            @pl.when(not_deg)
            def _():
                wait_one_gather(p, b)
                pltpu.async_copy(rows.at[p].at[b], acc.at[idst.at[p].at[b]],
                                 semS, add=True)

            @pl.when(is_deg)
            def _():
                pltpu.async_copy(ones, acc.at[idst.at[p].at[b]], semS,
                                 add=True)

        @pl.when(g > 0)
        def _():
            drain_scatters(1 - p)

        @pl.when(more)
        def _():
            issue_idst(g + 1, 1 - p)

            @pl.when(not_deg)
            def _():
                wait_isrc(1 - p)
                issue_gathers(1 - p)

    def pair(g2, _):
        group(g2 * 2, 0)
        group(g2 * 2 + 1, 1)
        return 0

    lax.fori_loop(0, ng // 2, pair, 0)
    drain_scatters(1)
    plsc.subcore_barrier()
    pltpu.sync_copy(acc.at[pl.ds(base_row, ROWS_PER_TILE)],
                    out_hbm.at[c].at[pl.ds(base_row, ROWS_PER_TILE)])


BLK = 2048
GRID = NP // BLK


def _row_spec(width):
    return pl.BlockSpec((BLK, width), lambda i: (i, 0))


def _pair_spec(width):
    return pl.BlockSpec((NC, BLK, width), lambda i: (0, i, 0))


_vec_spec = pl.BlockSpec((BLK,), lambda i: (i,))


def _full(a):
    return pl.BlockSpec(a.shape, lambda i: tuple(0 for _ in a.shape))


def _tc0_body(degp_ref, xp_ref, dinv_ref, g1_ref):
    deg = degp_ref[0, :, 0] + degp_ref[1, :, 0] + 1.0
    dinv = lax.rsqrt(deg)
    dinv_ref[...] = dinv
    g1_ref[...] = xp_ref[...] * dinv[:, None]


def _tc1_body(s1_ref, g1_ref, dinv_ref, w1_ref, b1_ref, w2_ref, g2_ref):
    dinv = dinv_ref[...]
    u = (s1_ref[0] + s1_ref[1] + g1_ref[...]) * dinv[:, None]
    h1 = jax.nn.relu(jnp.dot(u, w1_ref[...].T,
                             preferred_element_type=jnp.float32) + b1_ref[...])
    t2 = jnp.dot(h1, w2_ref[...].T, preferred_element_type=jnp.float32)
    g2 = t2 * dinv[:, None]
    g2_ref[0] = g2[:, :W]
    g2_ref[1] = g2[:, W:]


def _tc2_body(s2_ref, g2_ref, dinv_ref, b2_ref, w3_ref, g3_ref):
    dinv = dinv_ref[...]
    ya = (s2_ref[0] + g2_ref[0]) * dinv[:, None] + b2_ref[...][None, :W]
    yb = (s2_ref[1] + g2_ref[1]) * dinv[:, None] + b2_ref[...][None, W:]
    h2 = jax.nn.relu(jnp.concatenate([ya, yb], axis=1))
    t3 = jnp.dot(h2, w3_ref[...].T, preferred_element_type=jnp.float32)
    g3_ref[...] = t3 * dinv[:, None]


def _tc3_body(s3_ref, g3_ref, dinv_ref, b3_ref, w4_ref, g4_ref):
    dinv = dinv_ref[...]
    h3 = jax.nn.relu((s3_ref[0] + s3_ref[1] + g3_ref[...]) * dinv[:, None]
                     + b3_ref[...][None, :])
    t4 = jnp.dot(h3, w4_ref[...].T, preferred_element_type=jnp.float32)
    g4 = t4 * dinv[:, None]
    g4_ref[...] = jnp.concatenate(
        [g4, jnp.zeros((BLK, W - 8), jnp.float32)], axis=1)


def _tc4_body(s4_ref, g4_ref, dinv_ref, b4_ref, out_ref):
    dinv = dinv_ref[...]
    out_ref[...] = ((s4_ref[0, :, :8] + s4_ref[1, :, :8] + g4_ref[:, :8])
                    * dinv[:, None] + b4_ref[...][None, :])


def _params(mode):
    return jnp.full((1, W), mode, jnp.int32)


def kernel(x, edge_index, W1, b1, W2, b2, W3, b3, W4, b4):
    f32 = jnp.float32
    x_pad = jnp.zeros((NT, W), f32).at[:N_NODES, :11].set(x)
    pad = jnp.full((2, EP - N_EDGES), N_NODES, jnp.int32)
    epad = jnp.concatenate([edge_index.astype(jnp.int32), pad], axis=1)
    src2d = epad[0].reshape(NCHUNKS, CHUNK)
    dst2d = epad[1].reshape(NCHUNKS, CHUNK)
    W1p = jnp.pad(W1, ((0, 0), (0, W - 11)))

    degp = _sc_pass(_params(MODE_DEG), x_pad, src2d, dst2d)
    dinv, g1 = pl.pallas_call(
        _tc0_body,
        grid=(GRID,),
        in_specs=[_pair_spec(W), _row_spec(W)],
        out_specs=[_vec_spec, _row_spec(W)],
        out_shape=[jax.ShapeDtypeStruct((NP,), f32),
                   jax.ShapeDtypeStruct((NT, W), f32)],
    )(degp, x_pad)

    s1 = _sc_pass(_params(MODE_EDGE), g1, src2d, dst2d)
    g2 = pl.pallas_call(
        _tc1_body,
        grid=(GRID,),
        in_specs=[_pair_spec(W), _row_spec(W), _vec_spec,
                  _full(W1p), _full(b1), _full(W2)],
        out_specs=_pair_spec(W),
        out_shape=jax.ShapeDtypeStruct((NC, NP, W), f32),
    )(s1, g1, dinv, W1p, b1, W2)

    s2 = _sc_pass(_params(MODE_PANEL), g2.reshape(NT, W), src2d, dst2d)
    g3 = pl.pallas_call(
        _tc2_body,
        grid=(GRID,),
        in_specs=[_pair_spec(W), _pair_spec(W), _vec_spec,
                  _full(b2), _full(W3)],
        out_specs=_row_spec(W),
        out_shape=jax.ShapeDtypeStruct((NT, W), f32),
    )(s2, g2, dinv, b2, W3)

    s3 = _sc_pass(_params(MODE_EDGE), g3, src2d, dst2d)
    g4 = pl.pallas_call(
        _tc3_body,
        grid=(GRID,),
        in_specs=[_pair_spec(W), _row_spec(W), _vec_spec,
                  _full(b3), _full(W4)],
        out_specs=_row_spec(W),
        out_shape=jax.ShapeDtypeStruct((NT, W), f32),
    )(s3, g3, dinv, b3, W4)

    s4 = _sc_pass(_params(MODE_EDGE), g4, src2d, dst2d)
    out = pl.pallas_call(
        _tc4_body,
        grid=(GRID,),
        in_specs=[_pair_spec(W), _row_spec(W), _vec_spec, _full(b4)],
        out_specs=_row_spec(8),
        out_shape=jax.ShapeDtypeStruct((NP, 8), f32),
    )(s4, g4, dinv, b4)

    return out[:N_NODES]
